# Initial kernel scaffold; baseline (speedup 1.0000x reference)
import jax
import jax.numpy as jnp
from jax.experimental import pallas as pl


def kernel(x, edge_index, edge_attr, W_lin, b_lin, W1, b1, W2, b2):
    raise NotImplementedError("write your pallas kernel here")


# speedup vs baseline: 1.5510x; 1.5510x over previous
import functools

import jax
import jax.numpy as jnp
from jax import lax
from jax.experimental import pallas as pl
from jax.experimental.pallas import tpu as pltpu
from jax.experimental.pallas import tpu_sc as plsc

N = 50000
E = 800000
D = 100
DP = 128
C = 32
NCHUNK = DP // C
BLK = 256
KB = BLK // 128
NBLK = E // BLK
N_PAD = 50048
ROWS_PER_TILE = N_PAD // 16


def _edge_mm_body(ea_ref, w_ref, b_ref, out_ref):
    y = jnp.dot(ea_ref[...], w_ref[...],
                preferred_element_type=jnp.float32,
                precision=lax.Precision.HIGHEST) + b_ref[...]
    for c in range(NCHUNK):
        out_ref[c] = y[:, c * C:(c + 1) * C]


def _edge_mm(edge_attr, w_pad, b_pad):
    bk = 2000
    return pl.pallas_call(
        _edge_mm_body,
        grid=(E // bk,),
        in_specs=[
            pl.BlockSpec((bk, D), lambda i: (i, 0)),
            pl.BlockSpec((D, DP), lambda i: (0, 0)),
            pl.BlockSpec((1, DP), lambda i: (0, 0)),
        ],
        out_specs=pl.BlockSpec((NCHUNK, bk, C), lambda i: (0, i, 0)),
        out_shape=jax.ShapeDtypeStruct((NCHUNK, E, C), jnp.float32),
    )(edge_attr, w_pad, b_pad)


def _node_mm_body(x_ref, wi_ref, wj_ref, oi_ref, oj_ref):
    yi = jnp.dot(x_ref[...], wi_ref[...],
                 preferred_element_type=jnp.float32,
                 precision=lax.Precision.HIGHEST)
    yj = jnp.dot(x_ref[...], wj_ref[...],
                 preferred_element_type=jnp.float32,
                 precision=lax.Precision.HIGHEST)
    for c in range(NCHUNK):
        oi_ref[c] = yi[:, c * C:(c + 1) * C]
        oj_ref[c] = yj[:, c * C:(c + 1) * C]


def _node_mm(x, wi_pad, wj_pad):
    bn = 2000
    return pl.pallas_call(
        _node_mm_body,
        grid=(N // bn,),
        in_specs=[
            pl.BlockSpec((bn, D), lambda i: (i, 0)),
            pl.BlockSpec((D, DP), lambda i: (0, 0)),
            pl.BlockSpec((D, DP), lambda i: (0, 0)),
        ],
        out_specs=[
            pl.BlockSpec((NCHUNK, bn, C), lambda i: (0, i, 0)),
            pl.BlockSpec((NCHUNK, bn, C), lambda i: (0, i, 0)),
        ],
        out_shape=[
            jax.ShapeDtypeStruct((NCHUNK, N, C), jnp.float32),
            jax.ShapeDtypeStruct((NCHUNK, N, C), jnp.float32),
        ],
    )(x, wi_pad, wj_pad)


def _sc_body(dst_hbm, src_hbm, eaw_hbm, xiw_hbm, xjw_hbm, zeros_hbm,
             out_hbm, shared, dstb, srcb, dsta, srca, ebuf, abuf, bbuf,
             sem1, sem2):
    core = lax.axis_index("c")
    sid = lax.axis_index("s")

    def zero_table():
        pltpu.sync_copy(zeros_hbm,
                        shared.at[pl.ds(sid * ROWS_PER_TILE, ROWS_PER_TILE)])

    def edge_block(i, carry, *, chunk):
        blk = sid + 16 * i
        base = blk * BLK
        pltpu.sync_copy(dst_hbm.at[blk], dstb)
        pltpu.sync_copy(src_hbm.at[blk], srcb)
        cn = chunk * N
        for k in range(KB):
            for v in range(8):
                sl = pl.ds(v * 16, 16)
                dsta[k, sl] = dstb[k, sl] + cn
                srca[k, sl] = srcb[k, sl] + cn
        pltpu.sync_copy(eaw_hbm.at[pl.ds(chunk * E + base, BLK)], ebuf)
        cps = []
        for k in range(KB):
            rows = pl.ds(k * 128, 128)
            cps.append(pltpu.async_copy(xiw_hbm.at[dsta.at[k]],
                                        abuf.at[rows], sem1))
            cps.append(pltpu.async_copy(xjw_hbm.at[srca.at[k]],
                                        bbuf.at[rows], sem2))
        for cp in cps:
            cp.wait()

        def relu_row(r, acc):
            for h in range(2):
                sl = pl.ds(h * 16, 16)
                m = ebuf[r, sl] + abuf[r, sl] + bbuf[r, sl]
                ebuf[r, sl] = jnp.maximum(m, 0.0)
            return acc

        lax.fori_loop(0, BLK, relu_row, 0)
        for k in range(KB):
            rows = pl.ds(k * 128, 128)
            pltpu.sync_copy(ebuf.at[rows], shared.at[dstb.at[k]], add=True)
        return carry

    zero_table()
    plsc.subcore_barrier()
    for r in range(NCHUNK // 2):
        chunk = core * (NCHUNK // 2) + r
        nblk = (NBLK // 16) + jnp.where(sid < (NBLK % 16), 1, 0)
        lax.fori_loop(0, nblk,
                      functools.partial(edge_block, chunk=chunk), 0)
        plsc.subcore_barrier()
        row0 = sid * ROWS_PER_TILE
        pltpu.sync_copy(shared.at[pl.ds(row0, ROWS_PER_TILE)],
                        out_hbm.at[pl.ds(chunk * N_PAD + row0, ROWS_PER_TILE)])
        if r < NCHUNK // 2 - 1:
            zero_table()
        plsc.subcore_barrier()


def _sc_aggregate(dst3, src3, eaw, xiw, xjw, zeros):
    mesh = plsc.VectorSubcoreMesh(core_axis_name="c", subcore_axis_name="s")
    f = pl.kernel(
        _sc_body,
        mesh=mesh,
        out_type=jax.ShapeDtypeStruct((NCHUNK * N_PAD, C), jnp.float32),
        scratch_types=[
            pltpu.VMEM_SHARED((N_PAD, C), jnp.float32),
            pltpu.VMEM((KB, 128), jnp.int32),
            pltpu.VMEM((KB, 128), jnp.int32),
            pltpu.VMEM((KB, 128), jnp.int32),
            pltpu.VMEM((KB, 128), jnp.int32),
            pltpu.VMEM((BLK, C), jnp.float32),
            pltpu.VMEM((BLK, C), jnp.float32),
            pltpu.VMEM((BLK, C), jnp.float32),
            pltpu.SemaphoreType.DMA,
            pltpu.SemaphoreType.DMA,
        ],
        compiler_params=pltpu.CompilerParams(use_tc_tiling_on_sc=False),
    )
    return f(dst3, src3, eaw, xiw, xjw, zeros)


def _mlp_body(aggr_ref, x_ref, w1_ref, b1_ref, w2_ref, b2_ref, out_ref):
    cat = jnp.concatenate([aggr_ref[c] for c in range(NCHUNK)], axis=1)
    out_node = cat[:, :D] + x_ref[...]
    h = jax.nn.relu(jnp.dot(out_node, w1_ref[...],
                            preferred_element_type=jnp.float32,
                            precision=lax.Precision.HIGHEST) + b1_ref[...])
    out_ref[...] = jnp.dot(h, w2_ref[...],
                           preferred_element_type=jnp.float32,
                           precision=lax.Precision.HIGHEST) + b2_ref[...]


def _mlp(aggr4, x, w1t, b1, w2t, b2):
    bn = 2000
    return pl.pallas_call(
        _mlp_body,
        grid=(N // bn,),
        in_specs=[
            pl.BlockSpec((NCHUNK, bn, C), lambda i: (0, i, 0)),
            pl.BlockSpec((bn, D), lambda i: (i, 0)),
            pl.BlockSpec((D, D), lambda i: (0, 0)),
            pl.BlockSpec((1, D), lambda i: (0, 0)),
            pl.BlockSpec((D, D), lambda i: (0, 0)),
            pl.BlockSpec((1, D), lambda i: (0, 0)),
        ],
        out_specs=pl.BlockSpec((bn, D), lambda i: (i, 0)),
        out_shape=jax.ShapeDtypeStruct((N, D), jnp.float32),
    )(aggr4, x, w1t, b1, w2t, b2)


def kernel(x, edge_index, edge_attr, W_lin, b_lin, W1, b1, W2, b2):
    wi = W_lin[:, :D].T
    we = W_lin[:, D:2 * D].T
    wj = W_lin[:, 2 * D:].T
    pad = ((0, 0), (0, DP - D))
    wi_pad = jnp.pad(wi, pad)
    we_pad = jnp.pad(we, pad)
    wj_pad = jnp.pad(wj, pad)
    b_pad = jnp.pad(b_lin, (0, DP - D)).reshape(1, DP)

    eaw = _edge_mm(edge_attr, we_pad, b_pad).reshape(NCHUNK * E, C)
    xiw4, xjw4 = _node_mm(x, wi_pad, wj_pad)
    xiw = xiw4.reshape(NCHUNK * N, C)
    xjw = xjw4.reshape(NCHUNK * N, C)

    dst3 = edge_index[1].astype(jnp.int32).reshape(NBLK, KB, 128)
    src3 = edge_index[0].astype(jnp.int32).reshape(NBLK, KB, 128)
    zeros = jnp.zeros((ROWS_PER_TILE, C), jnp.float32)

    aggr = _sc_aggregate(dst3, src3, eaw, xiw, xjw, zeros)
    aggr4 = aggr.reshape(NCHUNK, N_PAD, C)

    return _mlp(aggr4, x, W1.T, b1.reshape(1, D), W2.T, b2.reshape(1, D))


# speedup vs baseline: 2.1729x; 1.4010x over previous
import functools

import jax
import jax.numpy as jnp
from jax import lax
from jax.experimental import pallas as pl
from jax.experimental.pallas import tpu as pltpu
from jax.experimental.pallas import tpu_sc as plsc

N = 50000
E = 800000
D = 100
DP = 128
C = 32
NCHUNK = DP // C
BLK = 256
KB = BLK // 128
NBLK = E // BLK
N_PAD = 50048
ROWS_PER_TILE = N_PAD // 16


def _edge_mm_body(ea_ref, w_ref, b_ref, out_ref):
    out_ref[...] = jnp.dot(ea_ref[...], w_ref[...],
                           preferred_element_type=jnp.float32,
                           precision=lax.Precision.HIGHEST) + b_ref[...]


def _edge_mm(edge_attr, w_pad, b_pad):
    bk = 2000
    return pl.pallas_call(
        _edge_mm_body,
        grid=(E // bk,),
        in_specs=[
            pl.BlockSpec((bk, D), lambda i: (i, 0)),
            pl.BlockSpec((D, DP), lambda i: (0, 0)),
            pl.BlockSpec((1, DP), lambda i: (0, 0)),
        ],
        out_specs=pl.BlockSpec((bk, DP), lambda i: (i, 0)),
        out_shape=jax.ShapeDtypeStruct((E, DP), jnp.float32),
    )(edge_attr, w_pad, b_pad)


def _node_mm_body(x_ref, wi_ref, wj_ref, oi_ref, oj_ref):
    yi = jnp.dot(x_ref[...], wi_ref[...],
                 preferred_element_type=jnp.float32,
                 precision=lax.Precision.HIGHEST)
    yj = jnp.dot(x_ref[...], wj_ref[...],
                 preferred_element_type=jnp.float32,
                 precision=lax.Precision.HIGHEST)
    for c in range(NCHUNK):
        oi_ref[c] = yi[:, c * C:(c + 1) * C]
        oj_ref[c] = yj[:, c * C:(c + 1) * C]


def _node_mm(x, wi_pad, wj_pad):
    bn = 2000
    return pl.pallas_call(
        _node_mm_body,
        grid=(N // bn,),
        in_specs=[
            pl.BlockSpec((bn, D), lambda i: (i, 0)),
            pl.BlockSpec((D, DP), lambda i: (0, 0)),
            pl.BlockSpec((D, DP), lambda i: (0, 0)),
        ],
        out_specs=[
            pl.BlockSpec((NCHUNK, bn, C), lambda i: (0, i, 0)),
            pl.BlockSpec((NCHUNK, bn, C), lambda i: (0, i, 0)),
        ],
        out_shape=[
            jax.ShapeDtypeStruct((NCHUNK, N, C), jnp.float32),
            jax.ShapeDtypeStruct((NCHUNK, N, C), jnp.float32),
        ],
    )(x, wi_pad, wj_pad)


def _sc_body(dst_hbm, src_hbm, eaw_hbm, xiw_hbm, xjw_hbm, zeros_hbm,
             out_hbm, shared, dstb, srcb, dsta, srca, ebuf, abuf, bbuf,
             sem1, sem2):
    core = lax.axis_index("c")
    sid = lax.axis_index("s")

    def zero_table():
        pltpu.sync_copy(zeros_hbm,
                        shared.at[pl.ds(sid * ROWS_PER_TILE, ROWS_PER_TILE)])

    def edge_block(i, carry, *, chunk):
        blk = sid + 16 * i
        base = blk * BLK
        pltpu.sync_copy(dst_hbm.at[blk], dstb)
        pltpu.sync_copy(src_hbm.at[blk], srcb)
        cn = chunk * N
        for k in range(KB):
            for v in range(8):
                sl = pl.ds(v * 16, 16)
                dsta[k, sl] = dstb[k, sl] + cn
                srca[k, sl] = srcb[k, sl] + cn
        pltpu.sync_copy(eaw_hbm.at[pl.ds(base, BLK), pl.ds(chunk * C, C)],
                        ebuf)
        cps = []
        for k in range(KB):
            rows = pl.ds(k * 128, 128)
            cps.append(pltpu.async_copy(xiw_hbm.at[dsta.at[k]],
                                        abuf.at[rows], sem1))
            cps.append(pltpu.async_copy(xjw_hbm.at[srca.at[k]],
                                        bbuf.at[rows], sem2))
        for cp in cps:
            cp.wait()

        def relu_row(r, acc):
            for h in range(2):
                sl = pl.ds(h * 16, 16)
                m = ebuf[r, sl] + abuf[r, sl] + bbuf[r, sl]
                ebuf[r, sl] = jnp.maximum(m, 0.0)
            return acc

        lax.fori_loop(0, BLK, relu_row, 0)
        for k in range(KB):
            rows = pl.ds(k * 128, 128)
            pltpu.sync_copy(ebuf.at[rows], shared.at[dstb.at[k]], add=True)
        return carry

    zero_table()
    plsc.subcore_barrier()
    for r in range(NCHUNK // 2):
        chunk = core * (NCHUNK // 2) + r
        nblk = (NBLK // 16) + jnp.where(sid < (NBLK % 16), 1, 0)
        lax.fori_loop(0, nblk,
                      functools.partial(edge_block, chunk=chunk), 0)
        plsc.subcore_barrier()
        row0 = sid * ROWS_PER_TILE
        pltpu.sync_copy(shared.at[pl.ds(row0, ROWS_PER_TILE)],
                        out_hbm.at[pl.ds(chunk * N_PAD + row0, ROWS_PER_TILE)])
        if r < NCHUNK // 2 - 1:
            zero_table()
        plsc.subcore_barrier()


def _sc_aggregate(dst3, src3, eaw, xiw, xjw, zeros):
    mesh = plsc.VectorSubcoreMesh(core_axis_name="c", subcore_axis_name="s")
    f = pl.kernel(
        _sc_body,
        mesh=mesh,
        out_type=jax.ShapeDtypeStruct((NCHUNK * N_PAD, C), jnp.float32),
        scratch_types=[
            pltpu.VMEM_SHARED((N_PAD, C), jnp.float32),
            pltpu.VMEM((KB, 128), jnp.int32),
            pltpu.VMEM((KB, 128), jnp.int32),
            pltpu.VMEM((KB, 128), jnp.int32),
            pltpu.VMEM((KB, 128), jnp.int32),
            pltpu.VMEM((BLK, C), jnp.float32),
            pltpu.VMEM((BLK, C), jnp.float32),
            pltpu.VMEM((BLK, C), jnp.float32),
            pltpu.SemaphoreType.DMA,
            pltpu.SemaphoreType.DMA,
        ],
        compiler_params=pltpu.CompilerParams(use_tc_tiling_on_sc=False),
    )
    return f(dst3, src3, eaw, xiw, xjw, zeros)


def _mlp_body(aggr_ref, x_ref, w1_ref, b1_ref, w2_ref, b2_ref, out_ref):
    cat = jnp.concatenate([aggr_ref[c] for c in range(NCHUNK)], axis=1)
    out_node = cat[:, :D] + x_ref[...]
    h = jax.nn.relu(jnp.dot(out_node, w1_ref[...],
                            preferred_element_type=jnp.float32,
                            precision=lax.Precision.HIGHEST) + b1_ref[...])
    out_ref[...] = jnp.dot(h, w2_ref[...],
                           preferred_element_type=jnp.float32,
                           precision=lax.Precision.HIGHEST) + b2_ref[...]


def _mlp(aggr4, x, w1t, b1, w2t, b2):
    bn = 2000
    return pl.pallas_call(
        _mlp_body,
        grid=(N // bn,),
        in_specs=[
            pl.BlockSpec((NCHUNK, bn, C), lambda i: (0, i, 0)),
            pl.BlockSpec((bn, D), lambda i: (i, 0)),
            pl.BlockSpec((D, D), lambda i: (0, 0)),
            pl.BlockSpec((1, D), lambda i: (0, 0)),
            pl.BlockSpec((D, D), lambda i: (0, 0)),
            pl.BlockSpec((1, D), lambda i: (0, 0)),
        ],
        out_specs=pl.BlockSpec((bn, D), lambda i: (i, 0)),
        out_shape=jax.ShapeDtypeStruct((N, D), jnp.float32),
    )(aggr4, x, w1t, b1, w2t, b2)


def kernel(x, edge_index, edge_attr, W_lin, b_lin, W1, b1, W2, b2):
    wi = W_lin[:, :D].T
    we = W_lin[:, D:2 * D].T
    wj = W_lin[:, 2 * D:].T
    pad = ((0, 0), (0, DP - D))
    wi_pad = jnp.pad(wi, pad)
    we_pad = jnp.pad(we, pad)
    wj_pad = jnp.pad(wj, pad)
    b_pad = jnp.pad(b_lin, (0, DP - D)).reshape(1, DP)

    eaw = _edge_mm(edge_attr, we_pad, b_pad)
    xiw4, xjw4 = _node_mm(x, wi_pad, wj_pad)
    xiw = xiw4.reshape(NCHUNK * N, C)
    xjw = xjw4.reshape(NCHUNK * N, C)

    dst3 = edge_index[1].astype(jnp.int32).reshape(NBLK, KB, 128)
    src3 = edge_index[0].astype(jnp.int32).reshape(NBLK, KB, 128)
    zeros = jnp.zeros((ROWS_PER_TILE, C), jnp.float32)

    aggr = _sc_aggregate(dst3, src3, eaw, xiw, xjw, zeros)
    aggr4 = aggr.reshape(NCHUNK, N_PAD, C)

    return _mlp(aggr4, x, W1.T, b1.reshape(1, D), W2.T, b2.reshape(1, D))


# speedup vs baseline: 2.8112x; 1.2937x over previous
import functools

import jax
import jax.numpy as jnp
from jax import lax
from jax.experimental import pallas as pl
from jax.experimental.pallas import tpu as pltpu
from jax.experimental.pallas import tpu_sc as plsc

N = 50000
E = 800000
D = 100
DP = 128
C = 32
NCHUNK = DP // C
BLK = 128
NBLK = E // BLK
N_PAD = 50048
ROWS_PER_TILE = N_PAD // 16


def _edge_mm_body(ea_ref, w_ref, b_ref, out_ref):
    out_ref[...] = jnp.dot(ea_ref[...], w_ref[...],
                           preferred_element_type=jnp.float32,
                           precision=lax.Precision.HIGHEST) + b_ref[...]


def _edge_mm(edge_attr, w_pad, b_pad):
    bk = 2000
    return pl.pallas_call(
        _edge_mm_body,
        grid=(E // bk,),
        in_specs=[
            pl.BlockSpec((bk, D), lambda i: (i, 0)),
            pl.BlockSpec((D, DP), lambda i: (0, 0)),
            pl.BlockSpec((1, DP), lambda i: (0, 0)),
        ],
        out_specs=pl.BlockSpec((bk, DP), lambda i: (i, 0)),
        out_shape=jax.ShapeDtypeStruct((E, DP), jnp.float32),
    )(edge_attr, w_pad, b_pad)


def _node_mm_body(x_ref, wi_ref, wj_ref, oi_ref, oj_ref):
    yi = jnp.dot(x_ref[...], wi_ref[...],
                 preferred_element_type=jnp.float32,
                 precision=lax.Precision.HIGHEST)
    yj = jnp.dot(x_ref[...], wj_ref[...],
                 preferred_element_type=jnp.float32,
                 precision=lax.Precision.HIGHEST)
    for c in range(NCHUNK):
        oi_ref[c] = yi[:, c * C:(c + 1) * C]
        oj_ref[c] = yj[:, c * C:(c + 1) * C]


def _node_mm(x, wi_pad, wj_pad):
    bn = 2000
    return pl.pallas_call(
        _node_mm_body,
        grid=(N // bn,),
        in_specs=[
            pl.BlockSpec((bn, D), lambda i: (i, 0)),
            pl.BlockSpec((D, DP), lambda i: (0, 0)),
            pl.BlockSpec((D, DP), lambda i: (0, 0)),
        ],
        out_specs=[
            pl.BlockSpec((NCHUNK, bn, C), lambda i: (0, i, 0)),
            pl.BlockSpec((NCHUNK, bn, C), lambda i: (0, i, 0)),
        ],
        out_shape=[
            jax.ShapeDtypeStruct((NCHUNK, N, C), jnp.float32),
            jax.ShapeDtypeStruct((NCHUNK, N, C), jnp.float32),
        ],
    )(x, wi_pad, wj_pad)


def _sc_body(idx_hbm, eaw_hbm, xiw_hbm, xjw_hbm, zeros_hbm,
             out_hbm, shared,
             idx0, idx1, adj0, adj1, e0, e1, a0, a1, b0, b1,
             se0, se1, sa0, sa1, sb0, sb1, ss0, ss1):
    core = lax.axis_index("c")
    sid = lax.axis_index("s")
    slots = ((idx0, adj0, e0, a0, b0, se0, sa0, sb0, ss0),
             (idx1, adj1, e1, a1, b1, se1, sa1, sb1, ss1))

    def zero_table():
        pltpu.sync_copy(zeros_hbm,
                        shared.at[pl.ds(sid * ROWS_PER_TILE, ROWS_PER_TILE)])

    def run_round(chunk):
        cn = chunk * N
        ccol = pl.ds(chunk * C, C)

        def issue(j, s, wait_scatter):
            idxb, adjb, eb, ab, bb, se, sa, sb, ss = slots[s]
            if wait_scatter:
                pltpu.make_async_copy(eb, shared.at[idxb.at[0]], ss).wait()
            blk = sid + 16 * j
            pltpu.sync_copy(idx_hbm.at[blk], idxb)
            for v in range(8):
                sl = pl.ds(v * 16, 16)
                adjb[0, sl] = idxb[0, sl] + cn
                adjb[1, sl] = idxb[1, sl] + cn
            pltpu.async_copy(eaw_hbm.at[pl.ds(blk * BLK, BLK), ccol], eb, se)
            pltpu.async_copy(xiw_hbm.at[adjb.at[0]], ab, sa)
            pltpu.async_copy(xjw_hbm.at[adjb.at[1]], bb, sb)

        def consume(s):
            idxb, adjb, eb, ab, bb, se, sa, sb, ss = slots[s]
            pltpu.make_async_copy(eaw_hbm.at[pl.ds(0, BLK), ccol], eb,
                                  se).wait()
            pltpu.make_async_copy(xiw_hbm.at[adjb.at[0]], ab, sa).wait()
            pltpu.make_async_copy(xjw_hbm.at[adjb.at[1]], bb, sb).wait()

            def relu_row(r, acc):
                for h in range(2):
                    sl = pl.ds(h * 16, 16)
                    m = eb[r, sl] + ab[r, sl] + bb[r, sl]
                    eb[r, sl] = jnp.maximum(m, 0.0)
                return acc

            lax.fori_loop(0, BLK, relu_row, 0)
            pltpu.async_copy(eb, shared.at[idxb.at[0]], ss, add=True)

        nblk = (NBLK // 16) + jnp.where(sid < (NBLK % 16), 1, 0)
        issue(0, 0, wait_scatter=False)
        issue(1, 1, wait_scatter=False)

        def pair(g, carry):
            consume(0)

            @pl.when(2 * g + 2 < nblk)
            def _():
                issue(2 * g + 2, 0, wait_scatter=True)

            consume(1)

            @pl.when(2 * g + 3 < nblk)
            def _():
                issue(2 * g + 3, 1, wait_scatter=True)

            return carry

        lax.fori_loop(0, nblk // 2, pair, 0)

        @pl.when(nblk % 2 == 1)
        def _():
            consume(0)

        for s in range(2):
            idxb, adjb, eb, ab, bb, se, sa, sb, ss = slots[s]
            pltpu.make_async_copy(eb, shared.at[idxb.at[0]], ss).wait()

    zero_table()
    plsc.subcore_barrier()
    for r in range(NCHUNK // 2):
        chunk = core * (NCHUNK // 2) + r
        run_round(chunk)
        plsc.subcore_barrier()
        row0 = sid * ROWS_PER_TILE
        pltpu.sync_copy(shared.at[pl.ds(row0, ROWS_PER_TILE)],
                        out_hbm.at[pl.ds(chunk * N_PAD + row0, ROWS_PER_TILE)])
        if r < NCHUNK // 2 - 1:
            zero_table()
        plsc.subcore_barrier()


def _sc_aggregate(idx3, eaw, xiw, xjw, zeros):
    mesh = plsc.VectorSubcoreMesh(core_axis_name="c", subcore_axis_name="s")
    f = pl.kernel(
        _sc_body,
        mesh=mesh,
        out_type=jax.ShapeDtypeStruct((NCHUNK * N_PAD, C), jnp.float32),
        scratch_types=(
            [pltpu.VMEM_SHARED((N_PAD, C), jnp.float32)]
            + [pltpu.VMEM((2, 128), jnp.int32) for _ in range(4)]
            + [pltpu.VMEM((BLK, C), jnp.float32) for _ in range(6)]
            + [pltpu.SemaphoreType.DMA for _ in range(8)]
        ),
        compiler_params=pltpu.CompilerParams(use_tc_tiling_on_sc=False),
    )
    return f(idx3, eaw, xiw, xjw, zeros)


def _mlp_body(aggr_ref, x_ref, w1_ref, b1_ref, w2_ref, b2_ref, out_ref):
    cat = jnp.concatenate([aggr_ref[c] for c in range(NCHUNK)], axis=1)
    out_node = cat[:, :D] + x_ref[...]
    h = jax.nn.relu(jnp.dot(out_node, w1_ref[...],
                            preferred_element_type=jnp.float32,
                            precision=lax.Precision.HIGHEST) + b1_ref[...])
    out_ref[...] = jnp.dot(h, w2_ref[...],
                           preferred_element_type=jnp.float32,
                           precision=lax.Precision.HIGHEST) + b2_ref[...]


def _mlp(aggr4, x, w1t, b1, w2t, b2):
    bn = 2000
    return pl.pallas_call(
        _mlp_body,
        grid=(N // bn,),
        in_specs=[
            pl.BlockSpec((NCHUNK, bn, C), lambda i: (0, i, 0)),
            pl.BlockSpec((bn, D), lambda i: (i, 0)),
            pl.BlockSpec((D, D), lambda i: (0, 0)),
            pl.BlockSpec((1, D), lambda i: (0, 0)),
            pl.BlockSpec((D, D), lambda i: (0, 0)),
            pl.BlockSpec((1, D), lambda i: (0, 0)),
        ],
        out_specs=pl.BlockSpec((bn, D), lambda i: (i, 0)),
        out_shape=jax.ShapeDtypeStruct((N, D), jnp.float32),
    )(aggr4, x, w1t, b1, w2t, b2)


def kernel(x, edge_index, edge_attr, W_lin, b_lin, W1, b1, W2, b2):
    wi = W_lin[:, :D].T
    we = W_lin[:, D:2 * D].T
    wj = W_lin[:, 2 * D:].T
    pad = ((0, 0), (0, DP - D))
    wi_pad = jnp.pad(wi, pad)
    we_pad = jnp.pad(we, pad)
    wj_pad = jnp.pad(wj, pad)
    b_pad = jnp.pad(b_lin, (0, DP - D)).reshape(1, DP)

    eaw = _edge_mm(edge_attr, we_pad, b_pad)
    xiw4, xjw4 = _node_mm(x, wi_pad, wj_pad)
    xiw = xiw4.reshape(NCHUNK * N, C)
    xjw = xjw4.reshape(NCHUNK * N, C)

    dst3 = edge_index[1].astype(jnp.int32).reshape(NBLK, 1, 128)
    src3 = edge_index[0].astype(jnp.int32).reshape(NBLK, 1, 128)
    idx3 = jnp.concatenate([dst3, src3], axis=1)
    zeros = jnp.zeros((ROWS_PER_TILE, C), jnp.float32)

    aggr = _sc_aggregate(idx3, eaw, xiw, xjw, zeros)
    aggr4 = aggr.reshape(NCHUNK, N_PAD, C)

    return _mlp(aggr4, x, W1.T, b1.reshape(1, D), W2.T, b2.reshape(1, D))
